# baseline (speedup 1.0000x reference)
import functools

import jax
import jax.numpy as jnp
from jax import lax
from jax.experimental import pallas as pl
from jax.experimental.pallas import tpu as pltpu
from jax.experimental.pallas import tpu_sc as plsc

BATCH = 16384
EMBED = 64
L = 16

_info = plsc.get_sparse_core_info()
NC, NS = _info.num_cores, _info.num_subcores
NW = NC * NS
BPW = BATCH // NW
GROUPS = BPW // L


def _mf_kernel(user_hbm, item_hbm, ue_hbm, ie_hbm, ub_hbm, ib_hbm, out_hbm,
               uidx_v, iidx_v, urows_v, irows_v, ub_v, ib_v, s_v, out_v,
               sem_u, sem_i, sem_ub, sem_ib):
    wid = lax.axis_index("s") * NC + lax.axis_index("c")
    base = wid * BPW

    pltpu.sync_copy(user_hbm.at[pl.ds(base, BPW)], uidx_v)
    pltpu.sync_copy(item_hbm.at[pl.ds(base, BPW)], iidx_v)

    cp_u = pltpu.async_copy(ue_hbm.at[uidx_v], urows_v, sem_u)
    cp_i = pltpu.async_copy(ie_hbm.at[iidx_v], irows_v, sem_i)
    cp_ub = pltpu.async_copy(ub_hbm.at[uidx_v], ub_v, sem_ub)
    cp_ib = pltpu.async_copy(ib_hbm.at[iidx_v], ib_v, sem_ib)
    cp_u.wait()
    cp_i.wait()
    cp_ub.wait()
    cp_ib.wait()

    lane = lax.iota(jnp.int32, L)
    _gdn = lax.GatherDimensionNumbers(
        offset_dims=(), collapsed_slice_dims=(0,), start_index_map=(0,))

    def _permute(p, idx):
        return lax.gather(p, idx[:, None], _gdn, slice_sizes=(1,),
                          mode=lax.GatherScatterMode.PROMISE_IN_BOUNDS)

    def group_body(g, carry):
        row0 = g * L
        t = jnp.zeros((L,), jnp.float32)
        for r in range(L):
            row = row0 + r
            p = urows_v[row, pl.ds(0, L)] * irows_v[row, pl.ds(0, L)]
            for c in range(1, EMBED // L):
                p = p + urows_v[row, pl.ds(c * L, L)] * irows_v[row, pl.ds(c * L, L)]
            for k in (8, 4, 2, 1):
                p = p + _permute(p, lane ^ k)
            t = jnp.where(lane == r, p, t)
        t = t + ub_v[pl.ds(row0, L)] + ib_v[pl.ds(row0, L)]
        out_v[pl.ds(row0, L)] = t
        return carry

    lax.fori_loop(0, GROUPS, group_body, 0)

    pltpu.sync_copy(out_v, out_hbm.at[pl.ds(base, BPW)])


@jax.jit
def _mf(user, item, user_e, item_e, ub_flat, ib_flat):
    mesh = plsc.VectorSubcoreMesh(core_axis_name="c", subcore_axis_name="s")
    return pl.kernel(
        _mf_kernel,
        mesh=mesh,
        out_type=jax.ShapeDtypeStruct((BATCH,), jnp.float32),
        compiler_params=pltpu.CompilerParams(use_tc_tiling_on_sc=False),
        scratch_types=[
            pltpu.VMEM((BPW,), jnp.int32),
            pltpu.VMEM((BPW,), jnp.int32),
            pltpu.VMEM((BPW, EMBED), jnp.float32),
            pltpu.VMEM((BPW, EMBED), jnp.float32),
            pltpu.VMEM((BPW,), jnp.float32),
            pltpu.VMEM((BPW,), jnp.float32),
            pltpu.VMEM((L * L,), jnp.float32),
            pltpu.VMEM((BPW,), jnp.float32),
            pltpu.SemaphoreType.DMA,
            pltpu.SemaphoreType.DMA,
            pltpu.SemaphoreType.DMA,
            pltpu.SemaphoreType.DMA,
        ],
    )(user, item, user_e, item_e, ub_flat, ib_flat)


def kernel(user, item, user_e, item_e, user_b, item_b):
    return _mf(user.astype(jnp.int32), item.astype(jnp.int32),
               user_e, item_e,
               user_b.reshape(-1), item_b.reshape(-1))
